# baseline (device time: 31041 ns/iter reference)
import jax
import jax.numpy as jnp
from jax import lax
from jax.experimental import pallas as pl
from jax.experimental.pallas import tpu as pltpu

N_DEV = 4
SCALE = 0.08838834764831843
DH = 128
N_BLK = 2


def kernel(x, Wq, Wo, Wk, Wv):
    _, sq, d_model = x.shape
    d_local = Wq.shape[1]
    n_heads_local = d_local // DH
    x2 = x.reshape(sq, d_model)

    def body(x_ref, wq_ref, wk_ref, wv_ref, wo_ref, out_ref,
             snd_ref, fl_ref, fr_ref, dg_ref, send_sems, recv_sems):
        my = lax.axis_index("i")
        left = lax.rem(my + N_DEV - 1, N_DEV)
        right = lax.rem(my + 1, N_DEV)
        diag = lax.rem(my + 2, N_DEV)

        barrier = pltpu.get_barrier_semaphore()
        for nbr in (left, right, diag):
            pl.semaphore_signal(
                barrier, inc=1,
                device_id=(nbr,), device_id_type=pl.DeviceIdType.MESH,
            )
        pl.semaphore_wait(barrier, 3)

        xb = x_ref[...].astype(jnp.bfloat16)
        wqb = wq_ref[...].astype(jnp.bfloat16)
        wob = wo_ref[...].astype(jnp.bfloat16)
        k = jnp.dot(xb, wk_ref[...].astype(jnp.bfloat16),
                    preferred_element_type=jnp.float32).astype(jnp.bfloat16)
        v = jnp.dot(xb, wv_ref[...].astype(jnp.bfloat16),
                    preferred_element_type=jnp.float32).astype(jnp.bfloat16)

        n_blk = N_BLK
        rows = sq // n_blk

        rdmas = []
        partials = []
        for b in range(n_blk):
            r0 = b * rows
            qb = jnp.dot(xb[r0:r0 + rows, :], wqb,
                         preferred_element_type=jnp.float32).astype(jnp.bfloat16)
            head_outs = []
            for h in range(n_heads_local):
                qh = qb[:, h * DH:(h + 1) * DH]
                kh = k[:, h * DH:(h + 1) * DH]
                vh = v[:, h * DH:(h + 1) * DH]
                s = lax.dot_general(
                    qh, kh, (((1,), (1,)), ((), ())),
                    preferred_element_type=jnp.float32,
                ) * SCALE
                m = jnp.max(s, axis=1, keepdims=True)
                p = jnp.exp(s - m)
                l = jnp.sum(p, axis=1, keepdims=True)
                o = jnp.dot(p.astype(jnp.bfloat16), vh,
                            preferred_element_type=jnp.float32)
                head_outs.append(o / l)
            attn_b = jnp.concatenate(head_outs, axis=1).astype(jnp.bfloat16)
            p_b = jnp.dot(attn_b, wob, preferred_element_type=jnp.float32)
            partials.append(p_b)
            snd_ref[r0:r0 + rows, :] = p_b.astype(jnp.bfloat16)

            blk = []
            for j, (tgt, dst) in enumerate(
                ((right, fl_ref), (left, fr_ref), (diag, dg_ref))
            ):
                rdma = pltpu.make_async_remote_copy(
                    src_ref=snd_ref.at[r0:r0 + rows, :],
                    dst_ref=dst.at[r0:r0 + rows, :],
                    send_sem=send_sems.at[3 * b + j],
                    recv_sem=recv_sems.at[3 * b + j],
                    device_id=(tgt,), device_id_type=pl.DeviceIdType.MESH,
                )
                rdma.start()
                blk.append(rdma)
            rdmas.append(blk)

        for b in range(n_blk):
            r0 = b * rows
            for rdma in rdmas[b]:
                rdma.wait_recv()
            out_ref[r0:r0 + rows, :] = (
                (partials[b] + fl_ref[r0:r0 + rows, :].astype(jnp.float32))
                + (fr_ref[r0:r0 + rows, :].astype(jnp.float32)
                   + dg_ref[r0:r0 + rows, :].astype(jnp.float32))
            )

        for blk in rdmas:
            for rdma in blk:
                rdma.wait_send()

    out = pl.pallas_call(
        body,
        out_shape=jax.ShapeDtypeStruct((sq, d_model), jnp.float32),
        in_specs=[pl.BlockSpec(memory_space=pltpu.VMEM)] * 5,
        out_specs=pl.BlockSpec(memory_space=pltpu.VMEM),
        scratch_shapes=[
            pltpu.VMEM((sq, d_model), jnp.bfloat16),
            pltpu.VMEM((sq, d_model), jnp.bfloat16),
            pltpu.VMEM((sq, d_model), jnp.bfloat16),
            pltpu.VMEM((sq, d_model), jnp.bfloat16),
            pltpu.SemaphoreType.DMA((3 * N_BLK,)),
            pltpu.SemaphoreType.DMA((3 * N_BLK,)),
        ],
        compiler_params=pltpu.CompilerParams(collective_id=0),
    )(x2, Wq, Wk, Wv, Wo)
    return out.reshape(1, sq, d_model)
